# baseline (speedup 1.0000x reference)
import functools

import jax
import jax.numpy as jnp
from jax import lax
from jax.experimental import pallas as pl
from jax.experimental.pallas import tpu as pltpu
from jax.experimental.pallas import tpu_sc as plsc

N = 10000
E = 160000
E_PAD = 163840
U_PAD = 4 * E_PAD
NW = 32
UT = U_PAD // NW
CH = 512
CHB = CH // 128
NCHUNK = UT // CH
V1 = N * 25
SB = 128


def _prep_body(row_ref, col_ref, p0x_ref, p0y_ref, p1x_ref, p1y_ref,
               bw0_ref, g1_ref, s1_ref, bw1_ref, g2_ref, s2_ref):
    pid = pl.program_id(0)
    row = row_ref[...]
    col = col_ref[...]
    r_i = lax.broadcasted_iota(jnp.int32, (16, 128), 0)
    l_i = lax.broadcasted_iota(jnp.int32, (16, 128), 1)
    e = (pid * 16 + r_i) * 128 + l_i
    valid = (e < E).astype(jnp.float32)

    def basis_wi(px, py):
        v0 = px * 4.0
        v1 = py * 4.0
        fl0 = jnp.floor(v0)
        fl1 = jnp.floor(v1)
        f0 = v0 - fl0
        f1 = v1 - fl1
        i0 = fl0.astype(jnp.int32)
        i1 = fl1.astype(jnp.int32)
        i0m = jnp.remainder(i0, 5)
        i0p = jnp.remainder(i0 + 1, 5)
        i1m = jnp.remainder(i1, 5)
        i1p = jnp.remainder(i1 + 1, 5)
        b = ((1 - f0) * (1 - f1), f0 * (1 - f1), (1 - f0) * f1, f0 * f1)
        wi = (i0m * 5 + i1m, i0p * 5 + i1m, i0m * 5 + i1p, i0p * 5 + i1p)
        return b, wi

    b0, wi0 = basis_wi(p0x_ref[...], p0y_ref[...])
    b1, wi1 = basis_wi(p1x_ref[...], p1y_ref[...])
    for s in range(4):
        bw0_ref[s] = b0[s] * valid
        s1_ref[s] = row * 25 + wi0[s]
        g1_ref[s] = col
        bw1_ref[s] = b1[s] * valid
        g2_ref[s] = col * 25 + wi1[s]
        s2_ref[s] = row


def _run_prep(row, col, p0x, p0y, p1x, p1y):
    eb = pl.BlockSpec((16, 128), lambda i: (i, 0))
    ub = pl.BlockSpec((4, 16, 128), lambda i: (0, i, 0))
    f32 = jnp.float32
    i32 = jnp.int32
    return pl.pallas_call(
        _prep_body,
        grid=(80,),
        in_specs=[eb] * 6,
        out_specs=[ub] * 6,
        out_shape=[jax.ShapeDtypeStruct((4, 1280, 128), d)
                   for d in (f32, i32, i32, f32, i32, i32)],
    )(row, col, p0x, p0y, p1x, p1y)


def _make_sc_accum(v_pad):
    vs = v_pad // 16
    mesh = plsc.VectorSubcoreMesh(core_axis_name="c", subcore_axis_name="s")

    @functools.partial(
        pl.kernel,
        out_type=jax.ShapeDtypeStruct((2 * v_pad, 8), jnp.float32),
        mesh=mesh,
        compiler_params=pltpu.CompilerParams(needs_layout_passes=False,
                                             use_tc_tiling_on_sc=False),
        scratch_types=[
            pltpu.VMEM_SHARED((v_pad, 8), jnp.float32),
            pltpu.VMEM((SB, 8), jnp.float32),
            pltpu.VMEM((SB, 8), jnp.float32),
            pltpu.VMEM((SB, 8), jnp.float32),
            pltpu.VMEM((CHB, 128), jnp.int32),
            pltpu.VMEM((CHB, 128), jnp.int32),
            pltpu.VMEM((CHB, 128), jnp.float32),
            pltpu.SemaphoreType.DMA,
            pltpu.SemaphoreType.DMA,
        ],
    )
    def sc_accum(table, gidx, sidx, bw, zeros, out, acc, gva, gvb, gvc,
                 giv, siv, bwv, gsem, ssem):
        cid = lax.axis_index("c")
        sid = lax.axis_index("s")
        wid = sid * 2 + cid
        ubase = wid * UT
        pltpu.sync_copy(zeros.at[pl.ds(sid * vs, vs)],
                        acc.at[pl.ds(sid * vs, vs)])
        plsc.subcore_barrier()

        lane = lax.broadcasted_iota(jnp.int32, (16,), 0)
        urow = lane // 8
        ucol = lane - urow * 8
        bufs = (gva, gvb, gvc)

        def compute(buf, j):
            for t in range(SB // 8):
                for q in range(4):
                    u = t * 8 + q * 2
                    rows = urow + u
                    g = plsc.load_gather(buf, [rows, ucol])
                    brow = jnp.full((16,), j, jnp.int32)
                    bcol = urow + u
                    bwp = plsc.load_gather(bwv, [brow, bcol])
                    plsc.store_scatter(buf, [rows, ucol], g * bwp)

        def chunk(k, carry):
            u0 = pl.multiple_of(ubase + k * CH, CH)
            r0 = pl.multiple_of(u0 // 128, CHB)
            pltpu.sync_copy(gidx.at[pl.ds(r0, CHB)], giv)
            pltpu.sync_copy(sidx.at[pl.ds(r0, CHB)], siv)
            pltpu.sync_copy(bw.at[pl.ds(r0, CHB)], bwv)
            def gath(j):
                return pltpu.async_copy(table.at[giv.at[j]], bufs[j % 3], gsem)

            def scat(j):
                return pltpu.async_copy(bufs[j % 3], acc.at[siv.at[j]], ssem,
                                        add=True)

            gd = {j: gath(j) for j in range(3)}
            sd = {}
            for j in range(CHB):
                gd[j].wait()
                compute(bufs[j % 3], j)
                if j >= 1 and j + 2 < CHB:
                    sd[j - 1].wait()
                    gd[j + 2] = gath(j + 2)
                sd[j] = scat(j)
            sd[CHB - 3].wait()
            sd[CHB - 2].wait()
            sd[CHB - 1].wait()
            return carry

        lax.fori_loop(0, NCHUNK, chunk, 0)
        plsc.subcore_barrier()
        pltpu.sync_copy(acc.at[pl.ds(sid * vs, vs)],
                        out.at[pl.ds(cid * v_pad + sid * vs, vs)])

    return sc_accum


def _mid_body(u0_ref, u1_ref, xp_ref, w_ref, rw_ref, wb_ref, dr_ref):
    z = jnp.dot(u0_ref[...] + u1_ref[...], w_ref[...],
                preferred_element_type=jnp.float32)
    deg = jnp.maximum(z[:, 32:33], 1.0)
    enc = z[:, :32] / deg + jnp.dot(xp_ref[...], rw_ref[...],
                                    preferred_element_type=jnp.float32)
    yr = jnp.dot(enc, wb_ref[...], preferred_element_type=jnp.float32)
    dr_ref[...] = jnp.concatenate(
        [yr, jnp.broadcast_to(deg, (deg.shape[0], 8))], axis=1)


def _run_mid(u0, u1, xp, w200d, rw8, wb):
    blk = 200
    return pl.pallas_call(
        _mid_body,
        grid=(N // blk,),
        in_specs=[
            pl.BlockSpec((blk, 200), lambda i: (i, 0)),
            pl.BlockSpec((blk, 200), lambda i: (i, 0)),
            pl.BlockSpec((blk, 8), lambda i: (i, 0)),
            pl.BlockSpec((200, 33), lambda i: (0, 0)),
            pl.BlockSpec((8, 32), lambda i: (0, 0)),
            pl.BlockSpec((32, 204), lambda i: (0, 0)),
        ],
        out_specs=pl.BlockSpec((blk, 212), lambda i: (i, 0)),
        out_shape=jax.ShapeDtypeStruct((N, 212), jnp.float32),
    )(u0, u1, xp, w200d, rw8, wb)


def _final_body(a0_ref, a1_ref, dr_ref, out_ref):
    s = a0_ref[...] + a1_ref[...]
    deg = dr_ref[:, 204:205]
    out_ref[...] = s[:, :4] / deg + dr_ref[:, 200:204]


def _run_final(a0, a1, dr):
    blk = 200
    return pl.pallas_call(
        _final_body,
        grid=(N // blk,),
        in_specs=[
            pl.BlockSpec((blk, 8), lambda i: (i, 0)),
            pl.BlockSpec((blk, 8), lambda i: (i, 0)),
            pl.BlockSpec((blk, 212), lambda i: (i, 0)),
        ],
        out_specs=pl.BlockSpec((blk, 4), lambda i: (i, 0)),
        out_shape=jax.ShapeDtypeStruct((N, 4), jnp.float32),
    )(a0, a1, dr)


def kernel(bipar_points, bipar_edges, pseudo0, pseudo1, weight, root_weight,
           weight1, root_weight1):
    f32 = jnp.float32
    x = bipar_points[0]
    row = bipar_edges[0, 0]
    col = bipar_edges[0, 1]

    def epad(a):
        return jnp.pad(a, (0, E_PAD - E)).reshape(1280, 128)

    bw0, g1, s1, bw1, g2, s2 = _run_prep(
        epad(row), epad(col),
        epad(pseudo0[:, 0]), epad(pseudo0[:, 1]),
        epad(pseudo1[:, 0]), epad(pseudo1[:, 1]))

    def uidx(a):
        return a.reshape(U_PAD // 128, 128)

    x_pad = jnp.concatenate(
        [x, jnp.ones((N, 1), f32), jnp.zeros((N, 4), f32)], axis=1)

    sc1 = _make_sc_accum(V1)
    u_out = sc1(x_pad, uidx(g1), uidx(s1), uidx(bw0),
                jnp.zeros((V1, 8), f32))
    u0 = u_out[:V1].reshape(N, 200)
    u1 = u_out[V1:].reshape(N, 200)

    w200 = jnp.pad(weight, ((0, 0), (0, 5), (0, 0))).reshape(200, 32)
    dcol = jnp.tile(jnp.array([0, 0, 0, 1, 0, 0, 0, 0], f32), 25)[:, None]
    w200d = jnp.concatenate([w200, dcol], axis=1)
    rw8 = jnp.pad(root_weight, ((0, 5), (0, 0)))
    w1p = jnp.pad(jnp.transpose(weight1, (1, 0, 2)),
                  ((0, 0), (0, 0), (0, 5))).reshape(32, 200)
    wb = jnp.concatenate([w1p, root_weight1,
                          jnp.zeros((32, 1), f32)], axis=1)

    dr = _run_mid(u0, u1, x_pad, w200d, rw8, wb)
    y2 = dr[:, :200].reshape(V1, 8)

    sc2 = _make_sc_accum(N)
    a_out = sc2(y2, uidx(g2), uidx(s2), uidx(bw1),
                jnp.zeros((N, 8), f32))
    a0 = a_out[:N]
    a1 = a_out[N:]

    out4 = _run_final(a0, a1, dr)
    return out4[:, :3]


# speedup vs baseline: 1.0062x; 1.0062x over previous
import functools

import jax
import jax.numpy as jnp
from jax import lax
from jax.experimental import pallas as pl
from jax.experimental.pallas import tpu as pltpu
from jax.experimental.pallas import tpu_sc as plsc

N = 10000
E = 160000
E_PAD = 163840
U_PAD = 4 * E_PAD
NW = 32
UT = U_PAD // NW
CH = 512
CHB = CH // 128
NCHUNK = UT // CH
V1 = N * 25
V1_SC = 250880
SB = 128


def _prep_body(row_ref, col_ref, p0x_ref, p0y_ref, p1x_ref, p1y_ref,
               bw0_ref, g1_ref, s1_ref, bw1_ref, g2_ref, s2_ref):
    pid = pl.program_id(0)
    row = row_ref[...]
    col = col_ref[...]
    r_i = lax.broadcasted_iota(jnp.int32, (16, 128), 0)
    l_i = lax.broadcasted_iota(jnp.int32, (16, 128), 1)
    e = (pid * 16 + r_i) * 128 + l_i
    valid = (e < E).astype(jnp.float32)

    def basis_wi(px, py):
        v0 = px * 4.0
        v1 = py * 4.0
        fl0 = jnp.floor(v0)
        fl1 = jnp.floor(v1)
        f0 = v0 - fl0
        f1 = v1 - fl1
        i0 = fl0.astype(jnp.int32)
        i1 = fl1.astype(jnp.int32)
        i0m = jnp.remainder(i0, 5)
        i0p = jnp.remainder(i0 + 1, 5)
        i1m = jnp.remainder(i1, 5)
        i1p = jnp.remainder(i1 + 1, 5)
        b = ((1 - f0) * (1 - f1), f0 * (1 - f1), (1 - f0) * f1, f0 * f1)
        wi = (i0m * 5 + i1m, i0p * 5 + i1m, i0m * 5 + i1p, i0p * 5 + i1p)
        return b, wi

    b0, wi0 = basis_wi(p0x_ref[...], p0y_ref[...])
    b1, wi1 = basis_wi(p1x_ref[...], p1y_ref[...])
    for s in range(4):
        bw0_ref[s] = b0[s] * valid
        s1_ref[s] = row * 25 + wi0[s]
        g1_ref[s] = col
        bw1_ref[s] = b1[s] * valid
        g2_ref[s] = col * 25 + wi1[s]
        s2_ref[s] = row


def _run_prep(row, col, p0x, p0y, p1x, p1y):
    eb = pl.BlockSpec((16, 128), lambda i: (i, 0))
    ub = pl.BlockSpec((4, 16, 128), lambda i: (0, i, 0))
    f32 = jnp.float32
    i32 = jnp.int32
    return pl.pallas_call(
        _prep_body,
        grid=(80,),
        in_specs=[eb] * 6,
        out_specs=[ub] * 6,
        out_shape=[jax.ShapeDtypeStruct((4, 1280, 128), d)
                   for d in (f32, i32, i32, f32, i32, i32)],
    )(row, col, p0x, p0y, p1x, p1y)


def _make_sc_accum(v_pad):
    vs = v_pad // 16
    mesh = plsc.VectorSubcoreMesh(core_axis_name="c", subcore_axis_name="s")
    out_shape = (2 * v_pad, 8)

    @functools.partial(
        pl.kernel,
        out_type=jax.ShapeDtypeStruct(out_shape, jnp.float32),
        mesh=mesh,
        compiler_params=pltpu.CompilerParams(needs_layout_passes=False,
                                             use_tc_tiling_on_sc=False),
        scratch_types=[
            pltpu.VMEM_SHARED((v_pad, 8), jnp.float32),
            pltpu.VMEM((SB, 8), jnp.float32),
            pltpu.VMEM((SB, 8), jnp.float32),
            pltpu.VMEM((SB, 8), jnp.float32),
            pltpu.VMEM((CHB, 128), jnp.int32),
            pltpu.VMEM((CHB, 128), jnp.int32),
            pltpu.VMEM((CHB, 128), jnp.float32),
            pltpu.SemaphoreType.DMA,
            pltpu.SemaphoreType.DMA,
        ],
    )
    def sc_accum(table, gidx, sidx, bw, zeros, out, acc, gva, gvb, gvc,
                 giv, siv, bwv, gsem, ssem):
        cid = lax.axis_index("c")
        sid = lax.axis_index("s")
        wid = sid * 2 + cid
        ubase = wid * UT
        pltpu.sync_copy(zeros.at[pl.ds(sid * vs, vs)],
                        acc.at[pl.ds(sid * vs, vs)])
        plsc.subcore_barrier()

        lane = lax.broadcasted_iota(jnp.int32, (16,), 0)
        urow = lane // 8
        ucol = lane - urow * 8
        bufs = (gva, gvb, gvc)

        def compute(buf, j):
            for t in range(SB // 8):
                for q in range(4):
                    u = t * 8 + q * 2
                    rows = urow + u
                    g = plsc.load_gather(buf, [rows, ucol])
                    brow = jnp.full((16,), j, jnp.int32)
                    bcol = urow + u
                    bwp = plsc.load_gather(bwv, [brow, bcol])
                    plsc.store_scatter(buf, [rows, ucol], g * bwp)

        def chunk(k, carry):
            u0 = pl.multiple_of(ubase + k * CH, CH)
            r0 = pl.multiple_of(u0 // 128, CHB)
            pltpu.sync_copy(gidx.at[pl.ds(r0, CHB)], giv)
            pltpu.sync_copy(sidx.at[pl.ds(r0, CHB)], siv)
            pltpu.sync_copy(bw.at[pl.ds(r0, CHB)], bwv)
            def gath(j):
                return pltpu.async_copy(table.at[giv.at[j]], bufs[j % 3], gsem)

            def scat(j):
                return pltpu.async_copy(bufs[j % 3], acc.at[siv.at[j]], ssem,
                                        add=True)

            gd = {j: gath(j) for j in range(3)}
            sd = {}
            for j in range(CHB):
                gd[j].wait()
                compute(bufs[j % 3], j)
                if j >= 1 and j + 2 < CHB:
                    sd[j - 1].wait()
                    gd[j + 2] = gath(j + 2)
                sd[j] = scat(j)
            sd[CHB - 3].wait()
            sd[CHB - 2].wait()
            sd[CHB - 1].wait()
            return carry

        lax.fori_loop(0, NCHUNK, chunk, 0)
        plsc.subcore_barrier()
        pltpu.sync_copy(acc.at[pl.ds(sid * vs, vs)],
                        out.at[pl.ds(cid * v_pad + sid * vs, vs)])

    return sc_accum


def _mid_body(u0_ref, u1_ref, xp_ref, w_ref, rw_ref, wb_ref, y2_ref, aux_ref):
    z = jnp.dot(u0_ref[...] + u1_ref[...], w_ref[...],
                preferred_element_type=jnp.float32)
    deg = jnp.maximum(z[:, 32:33], 1.0)
    enc = z[:, :32] / deg + jnp.dot(xp_ref[...], rw_ref[...],
                                    preferred_element_type=jnp.float32)
    yr = jnp.dot(enc, wb_ref[...], preferred_element_type=jnp.float32)
    y2_ref[...] = yr[:, :200]
    aux_ref[...] = jnp.concatenate(
        [yr[:, 200:204], jnp.broadcast_to(deg, (deg.shape[0], 4))], axis=1)


def _run_mid(u0, u1, xp, w200d, rw8, wb):
    blk = 200
    nb = N // blk
    return pl.pallas_call(
        _mid_body,
        grid=(nb,),
        in_specs=[
            pl.BlockSpec((blk, 200), lambda i: (i, 0)),
            pl.BlockSpec((blk, 200), lambda i: (i, 0)),
            pl.BlockSpec((blk, 8), lambda i: (i, 0)),
            pl.BlockSpec((200, 33), lambda i: (0, 0)),
            pl.BlockSpec((8, 32), lambda i: (0, 0)),
            pl.BlockSpec((32, 204), lambda i: (0, 0)),
        ],
        out_specs=[pl.BlockSpec((blk, 200), lambda i: (i, 0)),
                   pl.BlockSpec((blk, 8), lambda i: (i, 0))],
        out_shape=[jax.ShapeDtypeStruct((N, 200), jnp.float32),
                   jax.ShapeDtypeStruct((N, 8), jnp.float32)],
    )(u0, u1, xp, w200d, rw8, wb)


def _final_body(a0_ref, a1_ref, aux_ref, out_ref):
    s = a0_ref[...] + a1_ref[...]
    deg = aux_ref[:, 4:5]
    out_ref[...] = s[:, :4] / deg + aux_ref[:, :4]


def _run_final(a_out, aux):
    blk = 200
    nb = N // blk
    return pl.pallas_call(
        _final_body,
        grid=(nb,),
        in_specs=[
            pl.BlockSpec((blk, 8), lambda i: (i, 0)),
            pl.BlockSpec((blk, 8), lambda i: (i + nb, 0)),
            pl.BlockSpec((blk, 8), lambda i: (i, 0)),
        ],
        out_specs=pl.BlockSpec((blk, 4), lambda i: (i, 0)),
        out_shape=jax.ShapeDtypeStruct((N, 4), jnp.float32),
    )(a_out, a_out, aux)


def kernel(bipar_points, bipar_edges, pseudo0, pseudo1, weight, root_weight,
           weight1, root_weight1):
    f32 = jnp.float32
    x = bipar_points[0]
    row = bipar_edges[0, 0]
    col = bipar_edges[0, 1]

    def epad(a):
        return jnp.pad(a, (0, E_PAD - E)).reshape(1280, 128)

    bw0, g1, s1, bw1, g2, s2 = _run_prep(
        epad(row), epad(col),
        epad(pseudo0[:, 0]), epad(pseudo0[:, 1]),
        epad(pseudo1[:, 0]), epad(pseudo1[:, 1]))

    def uidx(a):
        return a.reshape(U_PAD // 128, 128)

    x_pad = jnp.concatenate(
        [x, jnp.ones((N, 1), f32), jnp.zeros((N, 4), f32)], axis=1)

    sc1 = _make_sc_accum(V1)
    u_out = sc1(x_pad, uidx(g1), uidx(s1), uidx(bw0),
                jnp.zeros((V1, 8), f32))
    u0 = u_out[:V1].reshape(N, 200)
    u1 = u_out[V1:].reshape(N, 200)

    w200 = jnp.pad(weight, ((0, 0), (0, 5), (0, 0))).reshape(200, 32)
    dcol = jnp.tile(jnp.array([0, 0, 0, 1, 0, 0, 0, 0], f32), 25)[:, None]
    w200d = jnp.concatenate([w200, dcol], axis=1)
    rw8 = jnp.pad(root_weight, ((0, 5), (0, 0)))
    w1p = jnp.pad(jnp.transpose(weight1, (1, 0, 2)),
                  ((0, 0), (0, 0), (0, 5))).reshape(32, 200)
    wb = jnp.concatenate([w1p, root_weight1,
                          jnp.zeros((32, 1), f32)], axis=1)

    y2, aux = _run_mid(u0, u1, x_pad, w200d, rw8, wb)

    sc2 = _make_sc_accum(N)
    a_out = sc2(y2.reshape(V1, 8), uidx(g2), uidx(s2), uidx(bw1),
                jnp.zeros((N, 8), f32))

    out4 = _run_final(a_out, aux)
    return out4[:, :3]


# speedup vs baseline: 1.0175x; 1.0112x over previous
import functools

import jax
import jax.numpy as jnp
from jax import lax
from jax.experimental import pallas as pl
from jax.experimental.pallas import tpu as pltpu
from jax.experimental.pallas import tpu_sc as plsc

N = 10000
E = 160000
E_PAD = 163840
U_PAD = 4 * E_PAD
NW = 32
UT = U_PAD // NW
CH = 1024
CHB = CH // 128
NCHUNK = UT // CH
V1 = N * 25
V1_SC = 250880
SB = 128


def _prep_body(row_ref, col_ref, p0x_ref, p0y_ref, p1x_ref, p1y_ref,
               bw0_ref, g1_ref, s1_ref, bw1_ref, g2_ref, s2_ref):
    pid = pl.program_id(0)
    row = row_ref[...]
    col = col_ref[...]
    r_i = lax.broadcasted_iota(jnp.int32, (16, 128), 0)
    l_i = lax.broadcasted_iota(jnp.int32, (16, 128), 1)
    e = (pid * 16 + r_i) * 128 + l_i
    valid = (e < E).astype(jnp.float32)

    def basis_wi(px, py):
        v0 = px * 4.0
        v1 = py * 4.0
        fl0 = jnp.floor(v0)
        fl1 = jnp.floor(v1)
        f0 = v0 - fl0
        f1 = v1 - fl1
        i0 = fl0.astype(jnp.int32)
        i1 = fl1.astype(jnp.int32)
        i0m = jnp.remainder(i0, 5)
        i0p = jnp.remainder(i0 + 1, 5)
        i1m = jnp.remainder(i1, 5)
        i1p = jnp.remainder(i1 + 1, 5)
        b = ((1 - f0) * (1 - f1), f0 * (1 - f1), (1 - f0) * f1, f0 * f1)
        wi = (i0m * 5 + i1m, i0p * 5 + i1m, i0m * 5 + i1p, i0p * 5 + i1p)
        return b, wi

    b0, wi0 = basis_wi(p0x_ref[...], p0y_ref[...])
    b1, wi1 = basis_wi(p1x_ref[...], p1y_ref[...])
    for s in range(4):
        bw0_ref[s] = b0[s] * valid
        s1_ref[s] = row * 25 + wi0[s]
        g1_ref[s] = col
        bw1_ref[s] = b1[s] * valid
        g2_ref[s] = col * 25 + wi1[s]
        s2_ref[s] = row


def _run_prep(row, col, p0x, p0y, p1x, p1y):
    eb = pl.BlockSpec((16, 128), lambda i: (i, 0))
    ub = pl.BlockSpec((4, 16, 128), lambda i: (0, i, 0))
    f32 = jnp.float32
    i32 = jnp.int32
    return pl.pallas_call(
        _prep_body,
        grid=(80,),
        in_specs=[eb] * 6,
        out_specs=[ub] * 6,
        out_shape=[jax.ShapeDtypeStruct((4, 1280, 128), d)
                   for d in (f32, i32, i32, f32, i32, i32)],
    )(row, col, p0x, p0y, p1x, p1y)


def _make_sc_accum(v_pad):
    vs = v_pad // 16
    mesh = plsc.VectorSubcoreMesh(core_axis_name="c", subcore_axis_name="s")
    out_shape = (2 * v_pad, 8)

    @functools.partial(
        pl.kernel,
        out_type=jax.ShapeDtypeStruct(out_shape, jnp.float32),
        mesh=mesh,
        compiler_params=pltpu.CompilerParams(needs_layout_passes=False,
                                             use_tc_tiling_on_sc=False),
        scratch_types=[
            pltpu.VMEM_SHARED((v_pad, 8), jnp.float32),
            pltpu.VMEM((SB, 8), jnp.float32),
            pltpu.VMEM((SB, 8), jnp.float32),
            pltpu.VMEM((CHB, 128), jnp.int32),
            pltpu.VMEM((CHB, 128), jnp.int32),
            pltpu.VMEM((CHB, 128), jnp.float32),
            pltpu.SemaphoreType.DMA,
            pltpu.SemaphoreType.DMA,
        ],
    )
    def sc_accum(table, gidx, sidx, bw, zeros, out, acc, gva, gvb,
                 giv, siv, bwv, gsem, ssem):
        cid = lax.axis_index("c")
        sid = lax.axis_index("s")
        wid = sid * 2 + cid
        ubase = wid * UT
        pltpu.sync_copy(zeros.at[pl.ds(sid * vs, vs)],
                        acc.at[pl.ds(sid * vs, vs)])
        plsc.subcore_barrier()

        lane = lax.broadcasted_iota(jnp.int32, (16,), 0)
        urow = lane // 8
        ucol = lane - urow * 8
        bufs = (gva, gvb)

        def compute(buf, j):
            for t in range(SB // 8):
                for q in range(4):
                    u = t * 8 + q * 2
                    rows = urow + u
                    g = plsc.load_gather(buf, [rows, ucol])
                    brow = jnp.full((16,), j, jnp.int32)
                    bcol = urow + u
                    bwp = plsc.load_gather(bwv, [brow, bcol])
                    plsc.store_scatter(buf, [rows, ucol], g * bwp)

        def chunk(k, carry):
            u0 = pl.multiple_of(ubase + k * CH, CH)
            r0 = pl.multiple_of(u0 // 128, CHB)
            pltpu.sync_copy(gidx.at[pl.ds(r0, CHB)], giv)
            pltpu.sync_copy(sidx.at[pl.ds(r0, CHB)], siv)
            pltpu.sync_copy(bw.at[pl.ds(r0, CHB)], bwv)
            def gath(j):
                return pltpu.async_copy(table.at[giv.at[j]], bufs[j % 2], gsem)

            def scat(j):
                return pltpu.async_copy(bufs[j % 2], acc.at[siv.at[j]], ssem,
                                        add=True)

            gd = {0: gath(0)}
            sd = {}
            for j in range(CHB):
                gd[j].wait()
                if j + 1 < CHB:
                    if j >= 1:
                        sd[j - 1].wait()
                    gd[j + 1] = gath(j + 1)
                compute(bufs[j % 2], j)
                sd[j] = scat(j)
            sd[CHB - 2].wait()
            sd[CHB - 1].wait()
            return carry

        lax.fori_loop(0, NCHUNK, chunk, 0)
        plsc.subcore_barrier()
        pltpu.sync_copy(acc.at[pl.ds(sid * vs, vs)],
                        out.at[pl.ds(cid * v_pad + sid * vs, vs)])

    return sc_accum


def _mid_body(u0_ref, u1_ref, xp_ref, w_ref, rw_ref, wb_ref, y2_ref, aux_ref):
    z = jnp.dot(u0_ref[...] + u1_ref[...], w_ref[...],
                preferred_element_type=jnp.float32)
    deg = jnp.maximum(z[:, 32:33], 1.0)
    enc = z[:, :32] / deg + jnp.dot(xp_ref[...], rw_ref[...],
                                    preferred_element_type=jnp.float32)
    yr = jnp.dot(enc, wb_ref[...], preferred_element_type=jnp.float32)
    y2_ref[...] = yr[:, :200]
    aux_ref[...] = jnp.concatenate(
        [yr[:, 200:204], jnp.broadcast_to(deg, (deg.shape[0], 4))], axis=1)


def _run_mid(u0, u1, xp, w200d, rw8, wb):
    blk = 200
    nb = N // blk
    return pl.pallas_call(
        _mid_body,
        grid=(nb,),
        in_specs=[
            pl.BlockSpec((blk, 200), lambda i: (i, 0)),
            pl.BlockSpec((blk, 200), lambda i: (i, 0)),
            pl.BlockSpec((blk, 8), lambda i: (i, 0)),
            pl.BlockSpec((200, 33), lambda i: (0, 0)),
            pl.BlockSpec((8, 32), lambda i: (0, 0)),
            pl.BlockSpec((32, 204), lambda i: (0, 0)),
        ],
        out_specs=[pl.BlockSpec((blk, 200), lambda i: (i, 0)),
                   pl.BlockSpec((blk, 8), lambda i: (i, 0))],
        out_shape=[jax.ShapeDtypeStruct((N, 200), jnp.float32),
                   jax.ShapeDtypeStruct((N, 8), jnp.float32)],
    )(u0, u1, xp, w200d, rw8, wb)


def _final_body(a0_ref, a1_ref, aux_ref, out_ref):
    s = a0_ref[...] + a1_ref[...]
    deg = aux_ref[:, 4:5]
    out_ref[...] = s[:, :4] / deg + aux_ref[:, :4]


def _run_final(a_out, aux):
    blk = 200
    nb = N // blk
    return pl.pallas_call(
        _final_body,
        grid=(nb,),
        in_specs=[
            pl.BlockSpec((blk, 8), lambda i: (i, 0)),
            pl.BlockSpec((blk, 8), lambda i: (i + nb, 0)),
            pl.BlockSpec((blk, 8), lambda i: (i, 0)),
        ],
        out_specs=pl.BlockSpec((blk, 4), lambda i: (i, 0)),
        out_shape=jax.ShapeDtypeStruct((N, 4), jnp.float32),
    )(a_out, a_out, aux)


def kernel(bipar_points, bipar_edges, pseudo0, pseudo1, weight, root_weight,
           weight1, root_weight1):
    f32 = jnp.float32
    x = bipar_points[0]
    row = bipar_edges[0, 0]
    col = bipar_edges[0, 1]

    def epad(a):
        return jnp.pad(a, (0, E_PAD - E)).reshape(1280, 128)

    bw0, g1, s1, bw1, g2, s2 = _run_prep(
        epad(row), epad(col),
        epad(pseudo0[:, 0]), epad(pseudo0[:, 1]),
        epad(pseudo1[:, 0]), epad(pseudo1[:, 1]))

    def uidx(a):
        return a.reshape(U_PAD // 128, 128)

    x_pad = jnp.concatenate(
        [x, jnp.ones((N, 1), f32), jnp.zeros((N, 4), f32)], axis=1)

    sc1 = _make_sc_accum(V1)
    u_out = sc1(x_pad, uidx(g1), uidx(s1), uidx(bw0),
                jnp.zeros((V1, 8), f32))
    u0 = u_out[:V1].reshape(N, 200)
    u1 = u_out[V1:].reshape(N, 200)

    w200 = jnp.pad(weight, ((0, 0), (0, 5), (0, 0))).reshape(200, 32)
    dcol = jnp.tile(jnp.array([0, 0, 0, 1, 0, 0, 0, 0], f32), 25)[:, None]
    w200d = jnp.concatenate([w200, dcol], axis=1)
    rw8 = jnp.pad(root_weight, ((0, 5), (0, 0)))
    w1p = jnp.pad(jnp.transpose(weight1, (1, 0, 2)),
                  ((0, 0), (0, 0), (0, 5))).reshape(32, 200)
    wb = jnp.concatenate([w1p, root_weight1,
                          jnp.zeros((32, 1), f32)], axis=1)

    y2, aux = _run_mid(u0, u1, x_pad, w200d, rw8, wb)

    sc2 = _make_sc_accum(N)
    a_out = sc2(y2.reshape(V1, 8), uidx(g2), uidx(s2), uidx(bw1),
                jnp.zeros((N, 8), f32))

    out4 = _run_final(a_out, aux)
    return out4[:, :3]
